# baseline (speedup 1.0000x reference)
import functools

import jax
import jax.numpy as jnp
from jax import lax
from jax.experimental import pallas as pl
from jax.experimental.pallas import tpu as pltpu
from jax.experimental.pallas import tpu_sc as plsc

N = 10000
E = 160000
D = 256
H = 128
NC = 2
NS = 16
LANES = 16
EC = 64
NCH = 80
NPAIR = NCH // 2
EPT = EC * NCH
E_PAD = EPT * NS * NC
GRP = EC // LANES
NROW = NS * NC * NCH

PERM = []
for _v in range(8):
    PERM += [32 * _v + 2 * _i for _i in range(16)]
    PERM += [32 * _v + 2 * _i + 1 for _i in range(16)]


def _sc_aggregate(xi, dst2, col2, w2):
    mesh = plsc.VectorSubcoreMesh(core_axis_name="c", subcore_axis_name="s")

    @functools.partial(
        pl.kernel,
        out_type=[jax.ShapeDtypeStruct((N, H), jnp.float32),
                  jax.ShapeDtypeStruct((N, H), jnp.float32),
                  jax.ShapeDtypeStruct((N, H), jnp.float32),
                  jax.ShapeDtypeStruct((N, H), jnp.float32),
                  jax.ShapeDtypeStruct((NROW, EC, H), jnp.float32)],
        mesh=mesh,
        scratch_types=[
            pltpu.VMEM((4, EC), jnp.int32),
            pltpu.VMEM((4, EC), jnp.int32),
            pltpu.VMEM((4, EC), jnp.float32),
            pltpu.VMEM((2, EC, H), jnp.int32),
            pltpu.VMEM((EC, H), jnp.float32),
            pltpu.VMEM((EC, H), jnp.float32),
            pltpu.VMEM_SHARED((N, H), jnp.float32),
            pltpu.SemaphoreType.DMA,
            pltpu.SemaphoreType.DMA,
            pltpu.SemaphoreType.DMA,
            pltpu.SemaphoreType.DMA,
            pltpu.SemaphoreType.DMA,
            pltpu.SemaphoreType.DMA,
            pltpu.SemaphoreType.DMA,
            pltpu.SemaphoreType.DMA,
        ],
    )
    def k(xh, dst_h, col_h, w_h, outL0, outL1, outR0, outR1, tmp_h,
          cring, dring, wring, ibuf, lbuf, rbuf, acc,
          gsem0, gsem1, ssemL, ssemR, dwsem0, dwsem1, csem0, csem1):
        c = lax.axis_index("c")
        s = lax.axis_index("s")
        wid = c * NS + s
        base = wid * NCH

        zv = jnp.zeros((LANES,), jnp.float32)

        def zero_acc():
            def zrow(e, _):
                for d2 in range(H // LANES):
                    lbuf[e, pl.ds(d2 * LANES, LANES)] = zv
                return 0
            lax.fori_loop(0, EC, zrow, 0)
            for t in range(NS):
                @pl.when(s == t)
                def _(t=t):
                    b = 624 * t
                    ln = 640 if t == NS - 1 else 624
                    for i in range(ln // EC):
                        pltpu.sync_copy(lbuf,
                                        acc.at[pl.ds(b + i * EC, EC)])
                    tail = ln % EC
                    if tail:
                        pltpu.sync_copy(
                            lbuf.at[pl.ds(0, tail)],
                            acc.at[pl.ds(b + (ln // EC) * EC, tail)])

        def flush(out_hbm):
            for t in range(NS):
                @pl.when(s == t)
                def _(t=t):
                    b = 624 * t
                    ln = 640 if t == NS - 1 else 624
                    pltpu.sync_copy(acc.at[pl.ds(b, ln)],
                                    out_hbm.at[pl.ds(b, ln)])

        def scale(ib, q):
            def group(g2, _):
                w16 = wring[q, pl.ds(g2 * LANES, LANES)]
                for l in range(LANES):
                    e = g2 * LANES + l
                    wv = w16[l]

                    def vloop(v, _, tgt, voff):
                        pv = ibuf[ib, e, pl.ds((v + voff) * LANES, LANES)]
                        fa = lax.bitcast_convert_type(
                            lax.shift_left(pv, 16), jnp.float32)
                        fb = lax.bitcast_convert_type(
                            pv & jnp.int32(-65536), jnp.float32)
                        c0 = 32 * v
                        tgt[e, pl.ds(c0, LANES)] = fa * wv
                        tgt[e, pl.ds(c0 + LANES, LANES)] = fb * wv
                        return 0
                    lax.fori_loop(
                        0, 4, functools.partial(vloop, tgt=lbuf, voff=0), 0)
                    lax.fori_loop(
                        0, 4, functools.partial(vloop, tgt=rbuf, voff=4), 0)
                return 0
            lax.fori_loop(0, GRP, group, 0)

        def gwait(b, gsem):
            pltpu.make_async_copy(xh.at[cring.at[0]], ibuf.at[b],
                                  gsem).wait()

        def cwait(q, csem):
            pltpu.make_async_copy(col_h.at[0], cring.at[q], csem).wait()

        def dwwait(q, dwsem):
            pltpu.make_async_copy(dst_h.at[0], dring.at[q], dwsem).wait()
            pltpu.make_async_copy(w_h.at[0], wring.at[q], dwsem).wait()

        def lwait():
            pltpu.make_async_copy(lbuf, acc.at[dring.at[0]], ssemL).wait()

        def rwait():
            pltpu.make_async_copy(rbuf, tmp_h.at[base], ssemR).wait()

        zero_acc()
        plsc.subcore_barrier()

        for m in range(3):
            pltpu.sync_copy(col_h.at[base + m], cring.at[m])
        pltpu.sync_copy(dst_h.at[base], dring.at[0])
        pltpu.sync_copy(w_h.at[base], wring.at[0])
        pltpu.async_copy(xh.at[cring.at[0]], ibuf.at[0], gsem0)

        def pair(g, _):
            j0 = 2 * g
            j1 = j0 + 1
            q0 = j0 % 4
            q1 = j1 % 4

            pltpu.async_copy(dst_h.at[base + j1], dring.at[q1], dwsem1)
            pltpu.async_copy(w_h.at[base + j1], wring.at[q1], dwsem1)

            @pl.when(g > 0)
            def _():
                cwait(q1, csem1)
            pltpu.async_copy(xh.at[cring.at[q1]], ibuf.at[1], gsem1)

            gwait(0, gsem0)
            @pl.when(j1 + 2 < NCH)
            def _():
                pltpu.async_copy(col_h.at[base + j1 + 2],
                                 cring.at[(j1 + 2) % 4], csem1)

            @pl.when(j0 + 4 < NCH)
            def _():
                pltpu.async_copy(col_h.at[base + j0 + 4],
                                 cring.at[(j0 + 4) % 4], csem0)

            @pl.when(g > 0)
            def _():
                dwwait(q0, dwsem0)
                lwait()
                rwait()
            scale(0, q0)

            @pl.when(g < NPAIR - 1)
            def _():
                nq = (j0 + 2) % 4
                pltpu.async_copy(dst_h.at[base + j0 + 2], dring.at[nq],
                                 dwsem0)
                pltpu.async_copy(w_h.at[base + j0 + 2], wring.at[nq],
                                 dwsem0)

                @pl.when(g > 0)
                def _():
                    cwait(nq, csem0)
                pltpu.async_copy(xh.at[cring.at[nq]], ibuf.at[0], gsem0)
            pltpu.async_copy(lbuf, acc.at[dring.at[q0]], ssemL, add=True)
            pltpu.async_copy(rbuf, tmp_h.at[base + j0], ssemR)

            gwait(1, gsem1)
            dwwait(q1, dwsem1)
            lwait()
            rwait()
            scale(1, q1)
            pltpu.async_copy(lbuf, acc.at[dring.at[q1]], ssemL, add=True)
            pltpu.async_copy(rbuf, tmp_h.at[base + j1], ssemR)
            return 0
        lax.fori_loop(0, NPAIR, pair, 0)
        lwait()
        rwait()

        plsc.subcore_barrier()

        @pl.when(c == 0)
        def _():
            flush(outL0)

        @pl.when(c == 1)
        def _():
            flush(outL1)

        plsc.subcore_barrier()
        zero_acc()
        plsc.subcore_barrier()

        def p2swait(buf_ref, ssem):
            pltpu.make_async_copy(buf_ref, acc.at[dring.at[0]], ssem).wait()

        def p2rdwait(buf_ref, gsem):
            pltpu.make_async_copy(tmp_h.at[base], buf_ref, gsem).wait()

        def dwait(q, dwsem):
            pltpu.make_async_copy(dst_h.at[0], dring.at[q], dwsem).wait()

        pltpu.sync_copy(dst_h.at[base], dring.at[0])
        pltpu.async_copy(tmp_h.at[base], lbuf, gsem0)

        def pair2(g, _):
            j0 = 2 * g
            j1 = j0 + 1
            q0 = j0 % 4
            q1 = j1 % 4

            @pl.when(g > 0)
            def _():
                p2swait(rbuf, ssemR)
            pltpu.async_copy(tmp_h.at[base + j1], rbuf, gsem1)
            pltpu.async_copy(dst_h.at[base + j1], dring.at[q1], dwsem1)

            p2rdwait(lbuf, gsem0)

            @pl.when(g > 0)
            def _():
                dwait(q0, dwsem0)
            pltpu.async_copy(lbuf, acc.at[dring.at[q0]], ssemL, add=True)

            p2rdwait(rbuf, gsem1)
            dwait(q1, dwsem1)
            p2swait(lbuf, ssemL)

            @pl.when(g < NPAIR - 1)
            def _():
                nq = (j0 + 2) % 4
                pltpu.async_copy(tmp_h.at[base + j0 + 2], lbuf, gsem0)
                pltpu.async_copy(dst_h.at[base + j0 + 2], dring.at[nq],
                                 dwsem0)
            pltpu.async_copy(rbuf, acc.at[dring.at[q1]], ssemR, add=True)
            return 0
        lax.fori_loop(0, NPAIR, pair2, 0)
        p2swait(rbuf, ssemR)

        plsc.subcore_barrier()

        @pl.when(c == 0)
        def _():
            flush(outR0)

        @pl.when(c == 1)
        def _():
            flush(outR1)

    return k(xi, dst2, col2, w2)


def _tc_matmul_relu(l0, l1, r0, r1, Wl, Wr):
    BM = 1000

    def body(l0r, l1r, r0r, r1r, wlr, wrr, o_ref):
        acc = jnp.dot(l0r[...] + l1r[...], wlr[...],
                      preferred_element_type=jnp.float32,
                      precision=lax.Precision.HIGHEST)
        acc = acc + jnp.dot(r0r[...] + r1r[...], wrr[...],
                            preferred_element_type=jnp.float32,
                            precision=lax.Precision.HIGHEST)
        o_ref[...] = jnp.maximum(acc, 0.0)

    a_spec = pl.BlockSpec((BM, H), lambda i: (i, 0))
    w_spec = pl.BlockSpec((H, D), lambda i: (0, 0))
    return pl.pallas_call(
        body,
        grid=(N // BM,),
        in_specs=[a_spec] * 4 + [w_spec] * 2,
        out_specs=pl.BlockSpec((BM, D), lambda i: (i, 0)),
        out_shape=jax.ShapeDtypeStruct((N, D), jnp.float32),
    )(l0, l1, r0, r1, Wl, Wr)


def kernel(input, edge_index, edge_weight, W):
    ei = edge_index.astype(jnp.int32)
    npad = E_PAD - E
    dst = jnp.concatenate([ei[0], jnp.zeros((npad,), jnp.int32)])
    col = jnp.concatenate([ei[1], jnp.zeros((npad,), jnp.int32)])
    ew = jnp.concatenate([edge_weight, jnp.zeros((npad,), jnp.float32)])
    dst2 = dst.reshape(NROW, EC)
    col2 = col.reshape(NROW, EC)
    w2 = ew.reshape(NROW, EC)
    xb = input.astype(jnp.bfloat16)
    xi = lax.bitcast_convert_type(xb.reshape(N, H, 2), jnp.int32)
    Wp = W[jnp.array(PERM, dtype=jnp.int32)]
    l0, l1, r0, r1, _ = _sc_aggregate(xi, dst2, col2, w2)
    return _tc_matmul_relu(l0, l1, r0, r1, Wp[:H], Wp[H:])


# speedup vs baseline: 1.5368x; 1.5368x over previous
import functools

import jax
import jax.numpy as jnp
from jax import lax
from jax.experimental import pallas as pl
from jax.experimental.pallas import tpu as pltpu
from jax.experimental.pallas import tpu_sc as plsc

N = 10000
E = 160000
D = 256
H = 128
NS = 16
LANES = 16
EC = 128
NCHUNK = 80
NPAIR = NCHUNK // 2
EPT = EC * NCHUNK
E_PAD = EPT * NS
GRP = EC // LANES


def _sc_aggregate(xL, xR, dst2, col3, w2):
    mesh = plsc.VectorSubcoreMesh(core_axis_name="c", subcore_axis_name="s")

    @functools.partial(
        pl.kernel,
        out_type=[jax.ShapeDtypeStruct((N, H), jnp.float32),
                  jax.ShapeDtypeStruct((N, H), jnp.float32)],
        mesh=mesh,
        scratch_types=[
            pltpu.VMEM((NCHUNK, EC), jnp.int32),
            pltpu.VMEM((2, EC), jnp.int32),
            pltpu.VMEM((2, EC), jnp.float32),
            pltpu.VMEM((2, EC, H), jnp.float32),
            pltpu.VMEM_SHARED((N, H), jnp.float32),
            pltpu.SemaphoreType.DMA,
            pltpu.SemaphoreType.DMA,
            pltpu.SemaphoreType.DMA,
            pltpu.SemaphoreType.DMA,
            pltpu.SemaphoreType.DMA,
            pltpu.SemaphoreType.DMA,
        ],
    )
    def k(xLh, xRh, dst_h, col_h, w_h, outL, outR,
          col_v, dring, wring, buf, acc,
          gsem0, gsem1, ssem0, ssem1, dwsem0, dwsem1):
        c = lax.axis_index("c")
        s = lax.axis_index("s")
        pltpu.sync_copy(col_h.at[s], col_v)
        base = s * NCHUNK

        zv = jnp.zeros((LANES,), jnp.float32)

        def zrow(e, _):
            for d2 in range(H // LANES):
                buf[0, e, pl.ds(d2 * LANES, LANES)] = zv
            return 0
        lax.fori_loop(0, EC, zrow, 0)
        for t in range(NS):
            @pl.when(s == t)
            def _(t=t):
                b = 624 * t
                ln = 640 if t == NS - 1 else 624
                for i in range(ln // EC):
                    pltpu.sync_copy(buf.at[0], acc.at[pl.ds(b + i * EC, EC)])
                tail = ln % EC
                if tail:
                    pltpu.sync_copy(
                        buf.at[0, pl.ds(0, tail)],
                        acc.at[pl.ds(b + (ln // EC) * EC, tail)])
        plsc.subcore_barrier()

        def run(xh):
            def scale(b):
                def group(g, _):
                    w16 = wring[b, pl.ds(g * LANES, LANES)]
                    for l in range(LANES):
                        e = g * LANES + l
                        w = w16[l]
                        for d2 in range(H // LANES):
                            sl = pl.ds(d2 * LANES, LANES)
                            buf[b, e, sl] = buf[b, e, sl] * w
                    return 0
                lax.fori_loop(0, GRP, group, 0)

            def gather_wait(b, gsem):
                pltpu.make_async_copy(xh.at[col_v.at[0]], buf.at[b],
                                      gsem).wait()

            def dw_wait(b, dwsem):
                pltpu.make_async_copy(dst_h.at[0], dring.at[b], dwsem).wait()
                pltpu.make_async_copy(w_h.at[0], wring.at[b], dwsem).wait()

            def scatter_wait(b, ssem):
                pltpu.make_async_copy(buf.at[b], acc.at[dring.at[b]],
                                      ssem).wait()

            pltpu.sync_copy(dst_h.at[base], dring.at[0])
            pltpu.sync_copy(w_h.at[base], wring.at[0])
            pltpu.async_copy(xh.at[col_v.at[0]], buf.at[0], gsem0)

            def pair(g, _):
                j0 = 2 * g
                j1 = j0 + 1

                @pl.when(g > 0)
                def _():
                    scatter_wait(1, ssem1)
                pltpu.async_copy(dst_h.at[base + j1], dring.at[1], dwsem1)
                pltpu.async_copy(w_h.at[base + j1], wring.at[1], dwsem1)
                pltpu.async_copy(xh.at[col_v.at[j1]], buf.at[1], gsem1)

                gather_wait(0, gsem0)

                @pl.when(g > 0)
                def _():
                    dw_wait(0, dwsem0)
                scale(0)
                pltpu.async_copy(buf.at[0], acc.at[dring.at[0]], ssem0,
                                 add=True)

                gather_wait(1, gsem1)
                dw_wait(1, dwsem1)
                scale(1)

                scatter_wait(0, ssem0)

                @pl.when(g < NPAIR - 1)
                def _():
                    pltpu.async_copy(dst_h.at[base + j0 + 2], dring.at[0],
                                     dwsem0)
                    pltpu.async_copy(w_h.at[base + j0 + 2], wring.at[0],
                                     dwsem0)
                    pltpu.async_copy(xh.at[col_v.at[j0 + 2]], buf.at[0],
                                     gsem0)
                pltpu.async_copy(buf.at[1], acc.at[dring.at[1]], ssem1,
                                 add=True)
                return 0
            lax.fori_loop(0, NPAIR, pair, 0)
            scatter_wait(1, ssem1)

        def flush(out_hbm):
            for t in range(NS):
                @pl.when(s == t)
                def _(t=t):
                    b = 624 * t
                    ln = 640 if t == NS - 1 else 624
                    pltpu.sync_copy(acc.at[pl.ds(b, ln)],
                                    out_hbm.at[pl.ds(b, ln)])

        @pl.when(c == 0)
        def _():
            run(xLh)

        @pl.when(c == 1)
        def _():
            run(xRh)

        plsc.subcore_barrier()

        @pl.when(c == 0)
        def _():
            flush(outL)

        @pl.when(c == 1)
        def _():
            flush(outR)

    return k(xL, xR, dst2, col3, w2)


def _tc_matmul_relu(aL, aR, Wt, Wb):
    BM = 1000

    def body(aL_ref, aR_ref, wt_ref, wb_ref, o_ref):
        acc = jnp.dot(aL_ref[...], wt_ref[...],
                      preferred_element_type=jnp.float32,
                      precision=lax.Precision.HIGHEST)
        acc = acc + jnp.dot(aR_ref[...], wb_ref[...],
                            preferred_element_type=jnp.float32,
                            precision=lax.Precision.HIGHEST)
        o_ref[...] = jnp.maximum(acc, 0.0)

    return pl.pallas_call(
        body,
        grid=(N // BM,),
        in_specs=[pl.BlockSpec((BM, H), lambda i: (i, 0)),
                  pl.BlockSpec((BM, H), lambda i: (i, 0)),
                  pl.BlockSpec((H, D), lambda i: (0, 0)),
                  pl.BlockSpec((H, D), lambda i: (0, 0))],
        out_specs=pl.BlockSpec((BM, D), lambda i: (i, 0)),
        out_shape=jax.ShapeDtypeStruct((N, D), jnp.float32),
    )(aL, aR, Wt, Wb)


def kernel(input, edge_index, edge_weight, W):
    ei = edge_index.astype(jnp.int32)
    npad = E_PAD - E
    dst = jnp.concatenate([ei[0], jnp.zeros((npad,), jnp.int32)])
    col = jnp.concatenate([ei[1], jnp.zeros((npad,), jnp.int32)])
    ew = jnp.concatenate([edge_weight, jnp.zeros((npad,), jnp.float32)])
    dst2 = dst.reshape(NS * NCHUNK, EC)
    col3 = col.reshape(NS, NCHUNK, EC)
    w2 = ew.reshape(NS * NCHUNK, EC)
    xL = input[:, :H]
    xR = input[:, H:]
    aggL, aggR = _sc_aggregate(xL, xR, dst2, col3, w2)
    return _tc_matmul_relu(aggL, aggR, W[:H], W[H:])
